# baseline (speedup 1.0000x reference)
import functools

import jax
import jax.numpy as jnp
from jax import lax
from jax.experimental import pallas as pl
from jax.experimental.pallas import tpu as pltpu
from jax.experimental.pallas import tpu_sc as plsc

STATE_SIZE = 1000000
ACTION_SIZE = 64
BATCH = 16384

NUM_CORES = 2
NUM_SUBCORES = 16
NUM_WORKERS = NUM_CORES * NUM_SUBCORES
ROWS_PER_WORKER = BATCH // NUM_WORKERS
IDX_CHUNK = 128
NUM_CHUNKS = ROWS_PER_WORKER // IDX_CHUNK


def _sc_body(prefs_hbm, states_hbm, out_hbm, idx_v, rows_v, sem):
    wid = lax.axis_index("s") * NUM_CORES + lax.axis_index("c")
    base = wid * ROWS_PER_WORKER

    pltpu.sync_copy(states_hbm.at[pl.ds(base, ROWS_PER_WORKER)], idx_v)

    copies = []
    for j in range(NUM_CHUNKS):
        copies.append(
            pltpu.async_copy(
                prefs_hbm.at[idx_v.at[pl.ds(j * IDX_CHUNK, IDX_CHUNK)]],
                rows_v.at[pl.ds(j * IDX_CHUNK, IDX_CHUNK)],
                sem,
            )
        )
    for c in copies:
        c.wait()

    lanes = lax.iota(jnp.int32, 16)
    perms = [lanes ^ k for k in (1, 2, 4, 8)]

    dnums = lax.GatherDimensionNumbers(
        offset_dims=(), collapsed_slice_dims=(0,), start_index_map=(0,))

    def shuffle(v, p):
        return lax.gather(v, p[:, None], dnums, (1,),
                          mode=lax.GatherScatterMode.PROMISE_IN_BOUNDS)

    def butterfly(v, op):
        for p in perms:
            v = op(v, shuffle(v, p))
        return v

    def row_body(i, carry):
        r0 = rows_v[i, pl.ds(0, 16)]
        r1 = rows_v[i, pl.ds(16, 16)]
        r2 = rows_v[i, pl.ds(32, 16)]
        r3 = rows_v[i, pl.ds(48, 16)]
        m = butterfly(jnp.maximum(jnp.maximum(r0, r1), jnp.maximum(r2, r3)),
                      jnp.maximum)
        e0 = jnp.exp(r0 - m)
        e1 = jnp.exp(r1 - m)
        e2 = jnp.exp(r2 - m)
        e3 = jnp.exp(r3 - m)
        inv = 1.0 / butterfly((e0 + e1) + (e2 + e3), jnp.add)
        rows_v[i, pl.ds(0, 16)] = e0 * inv
        rows_v[i, pl.ds(16, 16)] = e1 * inv
        rows_v[i, pl.ds(32, 16)] = e2 * inv
        rows_v[i, pl.ds(48, 16)] = e3 * inv
        return carry

    lax.fori_loop(0, ROWS_PER_WORKER, row_body, 0, unroll=4)

    pltpu.sync_copy(rows_v, out_hbm.at[pl.ds(base, ROWS_PER_WORKER)])


@jax.jit
def _policy(states, prefs):
    mesh = plsc.VectorSubcoreMesh(core_axis_name="c", subcore_axis_name="s")
    run = pl.kernel(
        _sc_body,
        mesh=mesh,
        compiler_params=pltpu.CompilerParams(use_tc_tiling_on_sc=False),
        out_type=jax.ShapeDtypeStruct((BATCH, ACTION_SIZE), jnp.float32),
        scratch_types=[
            pltpu.VMEM((ROWS_PER_WORKER,), jnp.int32),
            pltpu.VMEM((ROWS_PER_WORKER, ACTION_SIZE), jnp.float32),
            pltpu.SemaphoreType.DMA,
        ],
    )
    return run(prefs, states)


def kernel(states, prefs):
    return _policy(states, prefs)
